# initial kernel scaffold (unmeasured)
import jax
import jax.numpy as jnp
from jax import lax
from jax.experimental import pallas as pl
from jax.experimental.pallas import tpu as pltpu

N_DEV = 4


def kernel(x, w_mat):
    m_per, k = x.shape
    _, n_loc = w_mat.shape

    def body(x_ref, w_ref, out_ref, comm_ref, send_sems, recv_sems,
             amax_comm, amax_send_sems, amax_recv_sems):
        my = lax.axis_index("i")
        left = lax.rem(my - 1 + N_DEV, N_DEV)
        right = lax.rem(my + 1, N_DEV)

        barrier_sem = pltpu.get_barrier_semaphore()
        for nbr in (left, right):
            pl.semaphore_signal(
                barrier_sem, inc=1,
                device_id=(nbr,), device_id_type=pl.DeviceIdType.MESH,
            )
        pl.semaphore_wait(barrier_sem, 2)

        out_ref[pl.ds(my * m_per, m_per), :] = jnp.dot(
            x_ref[...], w_ref[...], preferred_element_type=jnp.float32
        )

        for h in range(N_DEV - 1):
            src = x_ref if h == 0 else comm_ref.at[(h - 1) % 2]
            slot = h % 2
            rdma = pltpu.make_async_remote_copy(
                src_ref=src,
                dst_ref=comm_ref.at[slot],
                send_sem=send_sems.at[slot],
                recv_sem=recv_sems.at[slot],
                device_id=(right,),
                device_id_type=pl.DeviceIdType.MESH,
            )
            rdma.start()
            rdma.wait()
            origin = lax.rem(my - (h + 1) + N_DEV, N_DEV)
            out_ref[pl.ds(origin * m_per, m_per), :] = jnp.dot(
                comm_ref[slot], w_ref[...], preferred_element_type=jnp.float32
            )

        amax = jnp.max(jnp.abs(out_ref[...]))
        amax_comm[0] = jnp.full((8, 128), amax, jnp.float32)
        for h in range(N_DEV - 1):
            s, r = h % 2, (h + 1) % 2
            rdma = pltpu.make_async_remote_copy(
                src_ref=amax_comm.at[s],
                dst_ref=amax_comm.at[r],
                send_sem=amax_send_sems.at[s],
                recv_sem=amax_recv_sems.at[r],
                device_id=(right,),
                device_id_type=pl.DeviceIdType.MESH,
            )
            rdma.start()
            rdma.wait()
            amax = jnp.maximum(amax, jnp.max(amax_comm[r]))

        scale = amax * (1.0 / 448.0)
        q = (out_ref[...] * (448.0 / amax)).astype(jnp.float8_e4m3fn)
        out_ref[...] = q.astype(jnp.float32) * scale

    return pl.pallas_call(
        body,
        out_shape=jax.ShapeDtypeStruct((N_DEV * m_per, n_loc), jnp.float32),
        in_specs=[
            pl.BlockSpec(memory_space=pltpu.VMEM),
            pl.BlockSpec(memory_space=pltpu.VMEM),
        ],
        out_specs=pl.BlockSpec(memory_space=pltpu.VMEM),
        scratch_shapes=[
            pltpu.VMEM((2, m_per, k), jnp.float32),
            pltpu.SemaphoreType.DMA((2,)),
            pltpu.SemaphoreType.DMA((2,)),
            pltpu.VMEM((2, 8, 128), jnp.float32),
            pltpu.SemaphoreType.DMA((2,)),
            pltpu.SemaphoreType.DMA((2,)),
        ],
        compiler_params=pltpu.CompilerParams(collective_id=0),
    )(x, w_mat)


# baseline (device time: 599225 ns/iter reference)
import jax
import jax.numpy as jnp
from jax import lax
from jax.experimental import pallas as pl
from jax.experimental.pallas import tpu as pltpu

N_DEV = 4


def kernel(x, w_mat):
    m_per, k = x.shape
    _, n_loc = w_mat.shape

    def body(x_ref, w_ref, out_ref, comm_ref, send_sems, recv_sems,
             amax_comm, amax_send_sems, amax_recv_sems):
        my = lax.axis_index("i")
        left = lax.rem(my - 1 + N_DEV, N_DEV)
        right = lax.rem(my + 1, N_DEV)

        barrier_sem = pltpu.get_barrier_semaphore()
        for nbr in (left, right):
            pl.semaphore_signal(
                barrier_sem, inc=1,
                device_id=(nbr,), device_id_type=pl.DeviceIdType.MESH,
            )
        pl.semaphore_wait(barrier_sem, 2)

        out_ref[pl.ds(my * m_per, m_per), :] = jnp.dot(
            x_ref[...], w_ref[...], preferred_element_type=jnp.float32
        )

        half = m_per // 2
        for half_i in range(2):
            for h in range(N_DEV - 1):
                src = (
                    x_ref.at[pl.ds(half_i * half, half), :]
                    if h == 0
                    else comm_ref.at[(h - 1) % 2]
                )
                slot = h % 2
                rdma = pltpu.make_async_remote_copy(
                    src_ref=src,
                    dst_ref=comm_ref.at[slot],
                    send_sem=send_sems.at[slot],
                    recv_sem=recv_sems.at[slot],
                    device_id=(right,),
                    device_id_type=pl.DeviceIdType.MESH,
                )
                rdma.start()
                rdma.wait()
                origin = lax.rem(my - (h + 1) + N_DEV, N_DEV)
                out_ref[pl.ds(origin * m_per + half_i * half, half), :] = jnp.dot(
                    comm_ref[slot], w_ref[...], preferred_element_type=jnp.float32
                )

        amax = jnp.max(jnp.abs(out_ref[...]))
        amax_comm[0] = jnp.full((8, 128), amax, jnp.float32)
        for h in range(N_DEV - 1):
            s, r = h % 2, (h + 1) % 2
            rdma = pltpu.make_async_remote_copy(
                src_ref=amax_comm.at[s],
                dst_ref=amax_comm.at[r],
                send_sem=amax_send_sems.at[s],
                recv_sem=amax_recv_sems.at[r],
                device_id=(right,),
                device_id_type=pl.DeviceIdType.MESH,
            )
            rdma.start()
            rdma.wait()
            amax = jnp.maximum(amax, jnp.max(amax_comm[r]))

        scale = amax * (1.0 / 448.0)
        q = (out_ref[...] * (448.0 / amax)).astype(jnp.float8_e4m3fn)
        out_ref[...] = q.astype(jnp.float32) * scale

    return pl.pallas_call(
        body,
        out_shape=jax.ShapeDtypeStruct((N_DEV * m_per, n_loc), jnp.float32),
        in_specs=[
            pl.BlockSpec(memory_space=pltpu.VMEM),
            pl.BlockSpec(memory_space=pltpu.VMEM),
        ],
        out_specs=pl.BlockSpec(memory_space=pltpu.VMEM),
        scratch_shapes=[
            pltpu.VMEM((2, m_per // 2, k), jnp.float32),
            pltpu.SemaphoreType.DMA((2,)),
            pltpu.SemaphoreType.DMA((2,)),
            pltpu.VMEM((2, 8, 128), jnp.float32),
            pltpu.SemaphoreType.DMA((2,)),
            pltpu.SemaphoreType.DMA((2,)),
        ],
        compiler_params=pltpu.CompilerParams(
            collective_id=0,
            vmem_limit_bytes=100 * 1024 * 1024,
        ),
    )(x, w_mat)


# device time: 144300 ns/iter; 4.1526x vs baseline; 4.1526x over previous
import jax
import jax.numpy as jnp
from jax import lax
from jax.experimental import pallas as pl
from jax.experimental.pallas import tpu as pltpu

N_DEV = 4
HOPS = N_DEV - 1


def kernel(x, w_mat):
    m_per, k = x.shape
    _, n_loc = w_mat.shape
    half = n_loc // 2

    def body(x_ref, w_ref, out_ref,
             comm_r, comm_l, ysend,
             wsend_r, wrecv_r, wsend_l, wrecv_l,
             ysend_sems, yrecv_sems,
             amax_comm, amax_send_sems, amax_recv_sems):
        my = lax.axis_index("i")
        left = lax.rem(my - 1 + N_DEV, N_DEV)
        right = lax.rem(my + 1, N_DEV)

        barrier_sem = pltpu.get_barrier_semaphore()
        for nbr in (left, right):
            pl.semaphore_signal(
                barrier_sem, inc=1,
                device_id=(nbr,), device_id_type=pl.DeviceIdType.MESH,
            )
        pl.semaphore_wait(barrier_sem, 2)

        comm_r[3] = w_ref[:, :half].astype(jnp.bfloat16)
        comm_l[3] = w_ref[:, half:].astype(jnp.bfloat16)

        def w_send(comm, send_sems, recv_sems, src_slot, dst_slot, dst_dev):
            r = pltpu.make_async_remote_copy(
                src_ref=comm.at[src_slot],
                dst_ref=comm.at[dst_slot],
                send_sem=send_sems.at[dst_slot],
                recv_sem=recv_sems.at[dst_slot],
                device_id=(dst_dev,),
                device_id_type=pl.DeviceIdType.MESH,
            )
            r.start()
            return r

        sends = []
        sends.append(w_send(comm_r, wsend_r, wrecv_r, 3, 0, right))
        sends.append(w_send(comm_l, wsend_l, wrecv_l, 3, 0, left))

        own = jnp.dot(x_ref[...], w_ref[...], preferred_element_type=jnp.float32)
        out_ref[pl.ds(my * m_per, m_per), :] = own
        amax = jnp.max(jnp.abs(own))

        y_sends = []

        def y_block(dir_is_right, s, wslice):
            nonlocal amax
            if dir_is_right:
                origin = lax.rem(my - (s + 1) + N_DEV, N_DEV)
                col0 = 0
                b = 2 * s
            else:
                origin = lax.rem(my + (s + 1), N_DEV)
                col0 = half
                b = 2 * s + 1
            y = jnp.dot(
                x_ref[...], wslice.astype(jnp.float32),
                preferred_element_type=jnp.float32,
            )
            amax = jnp.maximum(amax, jnp.max(jnp.abs(y)))
            slot = b % 3
            if b >= 3:
                pltpu.make_async_remote_copy(
                    src_ref=ysend.at[slot],
                    dst_ref=ysend.at[slot],
                    send_sem=ysend_sems.at[b - 3],
                    recv_sem=yrecv_sems.at[0],
                    device_id=(right,),
                    device_id_type=pl.DeviceIdType.MESH,
                ).wait_send()
            ysend[slot] = y
            r = pltpu.make_async_remote_copy(
                src_ref=ysend.at[slot],
                dst_ref=out_ref.at[pl.ds(my * m_per, m_per),
                                   pl.ds(col0, half)],
                send_sem=ysend_sems.at[b],
                recv_sem=yrecv_sems.at[my * 2 + (0 if dir_is_right else 1)],
                device_id=(origin,),
                device_id_type=pl.DeviceIdType.MESH,
            )
            r.start()
            y_sends.append((r, b))

        for s in range(HOPS):
            for dir_is_right, comm, ssem, rsem, dst in (
                (True, comm_r, wsend_r, wrecv_r, right),
                (False, comm_l, wsend_l, wrecv_l, left),
            ):
                pltpu.make_async_remote_copy(
                    src_ref=comm.at[s],
                    dst_ref=comm.at[s],
                    send_sem=ssem.at[s],
                    recv_sem=rsem.at[s],
                    device_id=(dst,),
                    device_id_type=pl.DeviceIdType.MESH,
                ).wait_recv()
                if s + 1 < HOPS:
                    sends.append(w_send(comm, ssem, rsem, s, s + 1, dst))
                y_block(dir_is_right, s, comm[s])

        amax_comm[0] = jnp.full((8, 128), amax, jnp.float32)
        for h in range(HOPS):
            sl, rl = h % 2, (h + 1) % 2
            rdma = pltpu.make_async_remote_copy(
                src_ref=amax_comm.at[sl],
                dst_ref=amax_comm.at[rl],
                send_sem=amax_send_sems.at[sl],
                recv_sem=amax_recv_sems.at[rl],
                device_id=(right,),
                device_id_type=pl.DeviceIdType.MESH,
            )
            rdma.start()
            rdma.wait()
            amax = jnp.maximum(amax, jnp.max(amax_comm[rl]))

        for d in range(1, N_DEV):
            i = lax.rem(my + d, N_DEV)
            for c in range(2):
                pltpu.make_async_remote_copy(
                    src_ref=ysend.at[0],
                    dst_ref=out_ref.at[pl.ds(i * m_per, m_per),
                                       pl.ds(c * half, half)],
                    send_sem=ysend_sems.at[0],
                    recv_sem=yrecv_sems.at[i * 2 + c],
                    device_id=(right,),
                    device_id_type=pl.DeviceIdType.MESH,
                ).wait_recv()

        scale = amax * (1.0 / 448.0)
        q = (out_ref[...] * (448.0 / amax)).astype(jnp.float8_e4m3fn)
        out_ref[...] = q.astype(jnp.float32) * scale

        for r in sends:
            r.wait_send()
        for r, b in y_sends:
            if b >= 3:
                r.wait_send()

    return pl.pallas_call(
        body,
        out_shape=jax.ShapeDtypeStruct((N_DEV * m_per, n_loc), jnp.float32),
        in_specs=[
            pl.BlockSpec(memory_space=pltpu.VMEM),
            pl.BlockSpec(memory_space=pltpu.VMEM),
        ],
        out_specs=pl.BlockSpec(memory_space=pltpu.VMEM),
        scratch_shapes=[
            pltpu.VMEM((4, k, half), jnp.bfloat16),
            pltpu.VMEM((4, k, half), jnp.bfloat16),
            pltpu.VMEM((3, m_per, half), jnp.float32),
            pltpu.SemaphoreType.DMA((HOPS,)),
            pltpu.SemaphoreType.DMA((HOPS,)),
            pltpu.SemaphoreType.DMA((HOPS,)),
            pltpu.SemaphoreType.DMA((HOPS,)),
            pltpu.SemaphoreType.DMA((2 * HOPS,)),
            pltpu.SemaphoreType.DMA((2 * N_DEV,)),
            pltpu.VMEM((2, 8, 128), jnp.float32),
            pltpu.SemaphoreType.DMA((2,)),
            pltpu.SemaphoreType.DMA((2,)),
        ],
        compiler_params=pltpu.CompilerParams(
            collective_id=0,
            vmem_limit_bytes=100 * 1024 * 1024,
        ),
    )(x, w_mat)


# device time: 124065 ns/iter; 4.8299x vs baseline; 1.1631x over previous
import jax
import jax.numpy as jnp
from jax import lax
from jax.experimental import pallas as pl
from jax.experimental.pallas import tpu as pltpu

N_DEV = 4
HOPS = N_DEV - 1


def kernel(x, w_mat):
    m_per, k = x.shape
    _, n_loc = w_mat.shape
    half = n_loc // 2

    def body(x_ref, w_ref, out_ref,
             comm_r, comm_l, ysend, yrecv,
             wsend_r, wrecv_r, wsend_l, wrecv_l,
             ysend_sems, yrecv_sems,
             amax_buf, amax_ssems, amax_rsems):
        my = lax.axis_index("i")
        left = lax.rem(my - 1 + N_DEV, N_DEV)
        right = lax.rem(my + 1, N_DEV)

        barrier_sem = pltpu.get_barrier_semaphore()
        for nbr in (left, right):
            pl.semaphore_signal(
                barrier_sem, inc=1,
                device_id=(nbr,), device_id_type=pl.DeviceIdType.MESH,
            )
        pl.semaphore_wait(barrier_sem, 2)

        comm_r[3] = w_ref[:, :half].astype(jnp.bfloat16)
        comm_l[3] = w_ref[:, half:].astype(jnp.bfloat16)

        def w_send(comm, send_sems, recv_sems, src_slot, dst_slot, dst_dev):
            r = pltpu.make_async_remote_copy(
                src_ref=comm.at[src_slot],
                dst_ref=comm.at[dst_slot],
                send_sem=send_sems.at[dst_slot],
                recv_sem=recv_sems.at[dst_slot],
                device_id=(dst_dev,),
                device_id_type=pl.DeviceIdType.MESH,
            )
            r.start()
            return r

        sends = []
        sends.append(w_send(comm_r, wsend_r, wrecv_r, 3, 0, right))
        sends.append(w_send(comm_l, wsend_l, wrecv_l, 3, 0, left))

        own = jnp.dot(x_ref[...], w_ref[...], preferred_element_type=jnp.float32)
        out_ref[pl.ds(my * m_per, m_per), :] = own
        amax = jnp.max(jnp.abs(own))

        y_sends = []

        def y_block(dir_is_right, s, wslice):
            nonlocal amax
            if dir_is_right:
                origin = lax.rem(my - (s + 1) + N_DEV, N_DEV)
                c = 0
                b = 2 * s
            else:
                origin = lax.rem(my + (s + 1), N_DEV)
                c = 1
                b = 2 * s + 1
            y = jnp.dot(
                x_ref[...], wslice.astype(jnp.float32),
                preferred_element_type=jnp.float32,
            )
            amax = jnp.maximum(amax, jnp.max(jnp.abs(y)))
            ysend[b] = y.astype(jnp.bfloat16)
            r = pltpu.make_async_remote_copy(
                src_ref=ysend.at[b],
                dst_ref=yrecv.at[2 * my + c],
                send_sem=ysend_sems.at[b],
                recv_sem=yrecv_sems.at[2 * my + c],
                device_id=(origin,),
                device_id_type=pl.DeviceIdType.MESH,
            )
            r.start()
            y_sends.append(r)

        for s in range(HOPS):
            for dir_is_right, comm, ssem, rsem, dst in (
                (True, comm_r, wsend_r, wrecv_r, right),
                (False, comm_l, wsend_l, wrecv_l, left),
            ):
                pltpu.make_async_remote_copy(
                    src_ref=comm.at[s],
                    dst_ref=comm.at[s],
                    send_sem=ssem.at[s],
                    recv_sem=rsem.at[s],
                    device_id=(dst,),
                    device_id_type=pl.DeviceIdType.MESH,
                ).wait_recv()
                if s + 1 < HOPS:
                    sends.append(w_send(comm, ssem, rsem, s, s + 1, dst))
                y_block(dir_is_right, s, comm[s])

        amax_buf[pl.ds(my, 1)] = jnp.full((1, 8, 128), amax, jnp.float32)
        for d in range(1, N_DEV):
            tgt = lax.rem(my + d, N_DEV)
            r = pltpu.make_async_remote_copy(
                src_ref=amax_buf.at[my],
                dst_ref=amax_buf.at[my],
                send_sem=amax_ssems.at[d],
                recv_sem=amax_rsems.at[my],
                device_id=(tgt,),
                device_id_type=pl.DeviceIdType.MESH,
            )
            r.start()
            sends.append(r)
        for d in range(1, N_DEV):
            src = lax.rem(my + d, N_DEV)
            pltpu.make_async_remote_copy(
                src_ref=amax_buf.at[src],
                dst_ref=amax_buf.at[src],
                send_sem=amax_ssems.at[0],
                recv_sem=amax_rsems.at[src],
                device_id=(right,),
                device_id_type=pl.DeviceIdType.MESH,
            ).wait_recv()
            amax = jnp.maximum(amax, jnp.max(amax_buf[pl.ds(src, 1)]))

        scale = amax * (1.0 / 448.0)
        inv = 448.0 / amax

        def quant(v):
            return (v * inv).astype(jnp.float8_e4m3fn).astype(jnp.float32) * scale

        out_ref[pl.ds(my * m_per, m_per), :] = quant(
            out_ref[pl.ds(my * m_per, m_per), :]
        )
        for d in range(1, N_DEV):
            i = lax.rem(my + d, N_DEV)
            for c in range(2):
                pltpu.make_async_remote_copy(
                    src_ref=ysend.at[0],
                    dst_ref=yrecv.at[2 * i + c],
                    send_sem=ysend_sems.at[0],
                    recv_sem=yrecv_sems.at[2 * i + c],
                    device_id=(right,),
                    device_id_type=pl.DeviceIdType.MESH,
                ).wait_recv()
                out_ref[pl.ds(i * m_per, m_per), pl.ds(c * half, half)] = quant(
                    yrecv[2 * i + c].astype(jnp.float32)
                )

        for r in sends:
            r.wait_send()
        for r in y_sends:
            r.wait_send()

    return pl.pallas_call(
        body,
        out_shape=jax.ShapeDtypeStruct((N_DEV * m_per, n_loc), jnp.float32),
        in_specs=[
            pl.BlockSpec(memory_space=pltpu.VMEM),
            pl.BlockSpec(memory_space=pltpu.VMEM),
        ],
        out_specs=pl.BlockSpec(memory_space=pltpu.VMEM),
        scratch_shapes=[
            pltpu.VMEM((4, k, half), jnp.bfloat16),
            pltpu.VMEM((4, k, half), jnp.bfloat16),
            pltpu.VMEM((2 * HOPS, m_per, half), jnp.bfloat16),
            pltpu.VMEM((2 * N_DEV, m_per, half), jnp.bfloat16),
            pltpu.SemaphoreType.DMA((HOPS,)),
            pltpu.SemaphoreType.DMA((HOPS,)),
            pltpu.SemaphoreType.DMA((HOPS,)),
            pltpu.SemaphoreType.DMA((HOPS,)),
            pltpu.SemaphoreType.DMA((2 * HOPS,)),
            pltpu.SemaphoreType.DMA((2 * N_DEV,)),
            pltpu.VMEM((N_DEV, 8, 128), jnp.float32),
            pltpu.SemaphoreType.DMA((N_DEV,)),
            pltpu.SemaphoreType.DMA((N_DEV,)),
        ],
        compiler_params=pltpu.CompilerParams(
            collective_id=0,
            vmem_limit_bytes=100 * 1024 * 1024,
        ),
    )(x, w_mat)


# device time: 119688 ns/iter; 5.0066x vs baseline; 1.0366x over previous
import jax
import jax.numpy as jnp
from jax import lax
from jax.experimental import pallas as pl
from jax.experimental.pallas import tpu as pltpu

N_DEV = 4
HOPS = N_DEV - 1


def kernel(x, w_mat):
    m_per, k = x.shape
    _, n_loc = w_mat.shape
    half = n_loc // 2

    def body(x_ref, w_ref, out_ref,
             comm_r, comm_l, yhold, ysend, yrecv,
             wsend_r, wrecv_r, wsend_l, wrecv_l,
             ysend_sems, yrecv_sems,
             amax_buf, amax_ssems, amax_rsems):
        my = lax.axis_index("i")
        left = lax.rem(my - 1 + N_DEV, N_DEV)
        right = lax.rem(my + 1, N_DEV)

        barrier_sem = pltpu.get_barrier_semaphore()
        for nbr in (left, right):
            pl.semaphore_signal(
                barrier_sem, inc=1,
                device_id=(nbr,), device_id_type=pl.DeviceIdType.MESH,
            )
        pl.semaphore_wait(barrier_sem, 2)

        comm_r[3] = w_ref[:, :half].astype(jnp.bfloat16)
        comm_l[3] = w_ref[:, half:].astype(jnp.bfloat16)

        def w_send(comm, send_sems, recv_sems, src_slot, dst_slot, dst_dev):
            r = pltpu.make_async_remote_copy(
                src_ref=comm.at[src_slot],
                dst_ref=comm.at[dst_slot],
                send_sem=send_sems.at[dst_slot],
                recv_sem=recv_sems.at[dst_slot],
                device_id=(dst_dev,),
                device_id_type=pl.DeviceIdType.MESH,
            )
            r.start()
            return r

        sends = []
        sends.append(w_send(comm_r, wsend_r, wrecv_r, 3, 0, right))
        sends.append(w_send(comm_l, wsend_l, wrecv_l, 3, 0, left))

        own = jnp.dot(x_ref[...], w_ref[...], preferred_element_type=jnp.float32)
        out_ref[pl.ds(my * m_per, m_per), :] = own
        amax = jnp.max(jnp.abs(own))

        for s in range(HOPS):
            for di, (comm, ssem, rsem, dst) in enumerate((
                (comm_r, wsend_r, wrecv_r, right),
                (comm_l, wsend_l, wrecv_l, left),
            )):
                pltpu.make_async_remote_copy(
                    src_ref=comm.at[s],
                    dst_ref=comm.at[s],
                    send_sem=ssem.at[s],
                    recv_sem=rsem.at[s],
                    device_id=(dst,),
                    device_id_type=pl.DeviceIdType.MESH,
                ).wait_recv()
                if s + 1 < HOPS:
                    sends.append(w_send(comm, ssem, rsem, s, s + 1, dst))
                y = jnp.dot(
                    x_ref[...], comm[s].astype(jnp.float32),
                    preferred_element_type=jnp.float32,
                )
                amax = jnp.maximum(amax, jnp.max(jnp.abs(y)))
                yhold[2 * s + di] = y.astype(jnp.bfloat16)

        amax_buf[pl.ds(my, 1)] = jnp.full((1, 8, 128), amax, jnp.float32)
        for d in range(1, N_DEV):
            tgt = lax.rem(my + d, N_DEV)
            r = pltpu.make_async_remote_copy(
                src_ref=amax_buf.at[my],
                dst_ref=amax_buf.at[my],
                send_sem=amax_ssems.at[d],
                recv_sem=amax_rsems.at[my],
                device_id=(tgt,),
                device_id_type=pl.DeviceIdType.MESH,
            )
            r.start()
            sends.append(r)
        for d in range(1, N_DEV):
            src = lax.rem(my + d, N_DEV)
            pltpu.make_async_remote_copy(
                src_ref=amax_buf.at[src],
                dst_ref=amax_buf.at[src],
                send_sem=amax_ssems.at[0],
                recv_sem=amax_rsems.at[src],
                device_id=(right,),
                device_id_type=pl.DeviceIdType.MESH,
            ).wait_recv()
            amax = jnp.maximum(amax, jnp.max(amax_buf[pl.ds(src, 1)]))

        scale = amax * (1.0 / 448.0)
        inv = 448.0 / amax

        y_sends = []
        for s in range(HOPS):
            for di in range(2):
                b = 2 * s + di
                if di == 0:
                    origin = lax.rem(my - (s + 1) + N_DEV, N_DEV)
                else:
                    origin = lax.rem(my + (s + 1), N_DEV)
                v = yhold[b].astype(jnp.float32) * inv
                v = jnp.clip(v, -448.0, 448.0)
                ysend[b] = v.astype(jnp.float8_e4m3fn)
                r = pltpu.make_async_remote_copy(
                    src_ref=ysend.at[b],
                    dst_ref=yrecv.at[2 * my + di],
                    send_sem=ysend_sems.at[b],
                    recv_sem=yrecv_sems.at[2 * my + di],
                    device_id=(origin,),
                    device_id_type=pl.DeviceIdType.MESH,
                )
                r.start()
                y_sends.append(r)

        q = (out_ref[pl.ds(my * m_per, m_per), :] * inv).astype(jnp.float8_e4m3fn)
        out_ref[pl.ds(my * m_per, m_per), :] = q.astype(jnp.float32) * scale

        for d in range(1, N_DEV):
            i = lax.rem(my + d, N_DEV)
            for c in range(2):
                pltpu.make_async_remote_copy(
                    src_ref=ysend.at[0],
                    dst_ref=yrecv.at[2 * i + c],
                    send_sem=ysend_sems.at[0],
                    recv_sem=yrecv_sems.at[2 * i + c],
                    device_id=(right,),
                    device_id_type=pl.DeviceIdType.MESH,
                ).wait_recv()
                out_ref[pl.ds(i * m_per, m_per), pl.ds(c * half, half)] = (
                    yrecv[2 * i + c].astype(jnp.float32) * scale
                )

        for r in sends:
            r.wait_send()
        for r in y_sends:
            r.wait_send()

    return pl.pallas_call(
        body,
        out_shape=jax.ShapeDtypeStruct((N_DEV * m_per, n_loc), jnp.float32),
        in_specs=[
            pl.BlockSpec(memory_space=pltpu.VMEM),
            pl.BlockSpec(memory_space=pltpu.VMEM),
        ],
        out_specs=pl.BlockSpec(memory_space=pltpu.VMEM),
        scratch_shapes=[
            pltpu.VMEM((4, k, half), jnp.bfloat16),
            pltpu.VMEM((4, k, half), jnp.bfloat16),
            pltpu.VMEM((2 * HOPS, m_per, half), jnp.bfloat16),
            pltpu.VMEM((2 * HOPS, m_per, half), jnp.float8_e4m3fn),
            pltpu.VMEM((2 * N_DEV, m_per, half), jnp.float8_e4m3fn),
            pltpu.SemaphoreType.DMA((HOPS,)),
            pltpu.SemaphoreType.DMA((HOPS,)),
            pltpu.SemaphoreType.DMA((HOPS,)),
            pltpu.SemaphoreType.DMA((HOPS,)),
            pltpu.SemaphoreType.DMA((2 * HOPS,)),
            pltpu.SemaphoreType.DMA((2 * N_DEV,)),
            pltpu.VMEM((N_DEV, 8, 128), jnp.float32),
            pltpu.SemaphoreType.DMA((N_DEV,)),
            pltpu.SemaphoreType.DMA((N_DEV,)),
        ],
        compiler_params=pltpu.CompilerParams(
            collective_id=0,
            vmem_limit_bytes=100 * 1024 * 1024,
        ),
    )(x, w_mat)


# device time: 109344 ns/iter; 5.4802x vs baseline; 1.0946x over previous
import jax
import jax.numpy as jnp
from jax import lax
from jax.experimental import pallas as pl
from jax.experimental.pallas import tpu as pltpu

N_DEV = 4
HOPS = N_DEV - 1


def kernel(x, w_mat):
    m_per, k = x.shape
    _, n_loc = w_mat.shape
    half = n_loc // 2

    def body(x_ref, w_ref, out_ref,
             xv, xbf, wv, own_buf, ostage,
             comm_r, comm_l, yhold, ysend, yrecv,
             ldma_sems, ostage_sems,
             wsend_r, wrecv_r, wsend_l, wrecv_l,
             ysend_sems, yrecv_sems,
             amax_buf, amax_ssems, amax_rsems):
        my = lax.axis_index("i")
        left = lax.rem(my - 1 + N_DEV, N_DEV)
        right = lax.rem(my + 1, N_DEV)

        w_dma_r = pltpu.make_async_copy(
            w_ref.at[:, pl.ds(0, half)], wv.at[:, pl.ds(0, half)],
            ldma_sems.at[0])
        w_dma_r.start()
        w_dma_l = pltpu.make_async_copy(
            w_ref.at[:, pl.ds(half, half)], wv.at[:, pl.ds(half, half)],
            ldma_sems.at[3])
        w_dma_l.start()
        x_dma0 = pltpu.make_async_copy(
            x_ref.at[pl.ds(0, m_per // 2), :], xv, ldma_sems.at[1])
        x_dma0.start()

        barrier_sem = pltpu.get_barrier_semaphore()
        for nbr in (left, right):
            pl.semaphore_signal(
                barrier_sem, inc=1,
                device_id=(nbr,), device_id_type=pl.DeviceIdType.MESH,
            )
        pl.semaphore_wait(barrier_sem, 2)

        qw = half // 2
        w_dma_r.wait()
        comm_r[6] = wv[:, 0 * qw:1 * qw].astype(jnp.bfloat16)
        comm_r[7] = wv[:, 1 * qw:2 * qw].astype(jnp.bfloat16)

        def w_send(comm, send_sems, recv_sems, src_slot, dst_slot, dst_dev):
            r = pltpu.make_async_remote_copy(
                src_ref=comm.at[src_slot],
                dst_ref=comm.at[dst_slot],
                send_sem=send_sems.at[dst_slot],
                recv_sem=recv_sems.at[dst_slot],
                device_id=(dst_dev,),
                device_id_type=pl.DeviceIdType.MESH,
            )
            r.start()
            return r

        sends = []
        sends.append(w_send(comm_r, wsend_r, wrecv_r, 6, 0, right))
        sends.append(w_send(comm_r, wsend_r, wrecv_r, 7, 1, right))
        w_dma_l.wait()
        comm_l[6] = wv[:, 2 * qw:3 * qw].astype(jnp.bfloat16)
        comm_l[7] = wv[:, 3 * qw:4 * qw].astype(jnp.bfloat16)
        sends.append(w_send(comm_l, wsend_l, wrecv_l, 6, 0, left))
        sends.append(w_send(comm_l, wsend_l, wrecv_l, 7, 1, left))

        x_dma0.wait()
        x_dma1 = pltpu.make_async_copy(
            x_ref.at[pl.ds(m_per // 2, m_per // 2), :], xv, ldma_sems.at[2])
        xbf[pl.ds(0, m_per // 2), :] = xv[...].astype(jnp.bfloat16)
        x_dma1.start()
        x_dma1.wait()
        xbf[pl.ds(m_per // 2, m_per // 2), :] = xv[...].astype(jnp.bfloat16)
        own = jnp.concatenate(
            [
                jnp.dot(xbf[...], comm_r[6], preferred_element_type=jnp.float32),
                jnp.dot(xbf[...], comm_r[7], preferred_element_type=jnp.float32),
                jnp.dot(xbf[...], comm_l[6], preferred_element_type=jnp.float32),
                jnp.dot(xbf[...], comm_l[7], preferred_element_type=jnp.float32),
            ],
            axis=1,
        )
        own_buf[...] = own
        amax = jnp.max(jnp.abs(own))

        for t in range(2 * HOPS):
            for di, (comm, ssem, rsem, dst) in enumerate((
                (comm_r, wsend_r, wrecv_r, right),
                (comm_l, wsend_l, wrecv_l, left),
            )):
                pltpu.make_async_remote_copy(
                    src_ref=comm.at[t],
                    dst_ref=comm.at[t],
                    send_sem=ssem.at[t],
                    recv_sem=rsem.at[t],
                    device_id=(dst,),
                    device_id_type=pl.DeviceIdType.MESH,
                ).wait_recv()
                if t + 2 < 2 * HOPS:
                    sends.append(w_send(comm, ssem, rsem, t, t + 2, dst))
                y = jnp.dot(
                    xbf[...], comm[t], preferred_element_type=jnp.float32
                )
                amax = jnp.maximum(amax, jnp.max(jnp.abs(y)))
                yhold[2 * (t // 2) + di, :, pl.ds((t % 2) * qw, qw)] = (
                    y.astype(jnp.bfloat16)
                )

        amax_buf[pl.ds(my, 1)] = jnp.full((1, 8, 128), amax, jnp.float32)
        for d in range(1, N_DEV):
            tgt = lax.rem(my + d, N_DEV)
            r = pltpu.make_async_remote_copy(
                src_ref=amax_buf.at[my],
                dst_ref=amax_buf.at[my],
                send_sem=amax_ssems.at[d],
                recv_sem=amax_rsems.at[my],
                device_id=(tgt,),
                device_id_type=pl.DeviceIdType.MESH,
            )
            r.start()
            sends.append(r)
        for d in range(1, N_DEV):
            src = lax.rem(my + d, N_DEV)
            pltpu.make_async_remote_copy(
                src_ref=amax_buf.at[src],
                dst_ref=amax_buf.at[src],
                send_sem=amax_ssems.at[0],
                recv_sem=amax_rsems.at[src],
                device_id=(right,),
                device_id_type=pl.DeviceIdType.MESH,
            ).wait_recv()
            amax = jnp.maximum(amax, jnp.max(amax_buf[pl.ds(src, 1)]))

        scale = amax * (1.0 / 448.0)
        inv = 448.0 / amax

        y_sends = []
        for s in range(HOPS):
            for di in range(2):
                b = 2 * s + di
                if di == 0:
                    origin = lax.rem(my - (s + 1) + N_DEV, N_DEV)
                else:
                    origin = lax.rem(my + (s + 1), N_DEV)
                v = yhold[b].astype(jnp.float32) * inv
                v = jnp.clip(v, -448.0, 448.0)
                ysend[b] = v.astype(jnp.float8_e4m3fn)
                r = pltpu.make_async_remote_copy(
                    src_ref=ysend.at[b],
                    dst_ref=yrecv.at[2 * my + di],
                    send_sem=ysend_sems.at[b],
                    recv_sem=yrecv_sems.at[2 * my + di],
                    device_id=(origin,),
                    device_id_type=pl.DeviceIdType.MESH,
                )
                r.start()
                y_sends.append(r)

        out_dmas = []

        def out_write(slot_i, row_dev, values):
            if len(out_dmas) >= 2:
                out_dmas[-2].wait()
            ostage[slot_i % 2] = values
            dma = pltpu.make_async_copy(
                ostage.at[slot_i % 2],
                out_ref.at[pl.ds(row_dev * m_per, m_per), :],
                ostage_sems.at[slot_i % 2],
            )
            dma.start()
            out_dmas.append(dma)

        q = (own_buf[...] * inv).astype(jnp.float8_e4m3fn)
        out_write(0, my, q.astype(jnp.float32) * scale)

        for d in range(1, N_DEV):
            i = lax.rem(my + d, N_DEV)
            for c in range(2):
                pltpu.make_async_remote_copy(
                    src_ref=ysend.at[0],
                    dst_ref=yrecv.at[2 * i + c],
                    send_sem=ysend_sems.at[0],
                    recv_sem=yrecv_sems.at[2 * i + c],
                    device_id=(right,),
                    device_id_type=pl.DeviceIdType.MESH,
                ).wait_recv()
            out_write(
                d, i,
                jnp.concatenate(
                    [yrecv[2 * i].astype(jnp.float32),
                     yrecv[2 * i + 1].astype(jnp.float32)],
                    axis=1,
                ) * scale,
            )

        for dma in out_dmas[-2:]:
            dma.wait()
        for r in sends:
            r.wait_send()
        for r in y_sends:
            r.wait_send()

    return pl.pallas_call(
        body,
        out_shape=jax.ShapeDtypeStruct((N_DEV * m_per, n_loc), jnp.float32),
        in_specs=[
            pl.BlockSpec(memory_space=pl.ANY),
            pl.BlockSpec(memory_space=pl.ANY),
        ],
        out_specs=pl.BlockSpec(memory_space=pl.ANY),
        scratch_shapes=[
            pltpu.VMEM((m_per // 2, k), jnp.float32),
            pltpu.VMEM((m_per, k), jnp.bfloat16),
            pltpu.VMEM((k, n_loc), jnp.float32),
            pltpu.VMEM((m_per, n_loc), jnp.float32),
            pltpu.VMEM((2, m_per, n_loc), jnp.float32),
            pltpu.VMEM((8, k, half // 2), jnp.bfloat16),
            pltpu.VMEM((8, k, half // 2), jnp.bfloat16),
            pltpu.VMEM((2 * HOPS, m_per, half), jnp.bfloat16),
            pltpu.VMEM((2 * HOPS, m_per, half), jnp.float8_e4m3fn),
            pltpu.VMEM((2 * N_DEV, m_per, half), jnp.float8_e4m3fn),
            pltpu.SemaphoreType.DMA((4,)),
            pltpu.SemaphoreType.DMA((2,)),
            pltpu.SemaphoreType.DMA((2 * HOPS,)),
            pltpu.SemaphoreType.DMA((2 * HOPS,)),
            pltpu.SemaphoreType.DMA((2 * HOPS,)),
            pltpu.SemaphoreType.DMA((2 * HOPS,)),
            pltpu.SemaphoreType.DMA((2 * HOPS,)),
            pltpu.SemaphoreType.DMA((2 * N_DEV,)),
            pltpu.VMEM((N_DEV, 8, 128), jnp.float32),
            pltpu.SemaphoreType.DMA((N_DEV,)),
            pltpu.SemaphoreType.DMA((N_DEV,)),
        ],
        compiler_params=pltpu.CompilerParams(
            collective_id=0,
            vmem_limit_bytes=100 * 1024 * 1024,
        ),
    )(x, w_mat)
